# baseline (speedup 1.0000x reference)
import functools

import jax
import jax.numpy as jnp
from jax import lax
from jax.experimental import pallas as pl
from jax.experimental.pallas import tpu as pltpu
from jax.experimental.pallas import tpu_sc as plsc

K = 16
N = 8192
E = 64
RB = 256
CB = 2048
NCH = N // CB


def _topk_body(xe_ref, ye_ref, score_ref, idx_ref, xp_ref):
    xb = xe_ref[...]
    xp_ref[...] = jnp.concatenate(
        [xb, jnp.zeros((RB, 128 - E), jnp.float32)], axis=1)
    xq = jnp.sum(xb * xb, axis=1, keepdims=True)

    BIG = jnp.int32(2 ** 30)
    INF = jnp.float32(jnp.inf)
    kio = lax.broadcasted_iota(jnp.int32, (RB, K), 1)
    MAJ = 16
    LAN = CB // MAJ

    def extract_chunk(d, c):
        dv = d.reshape(RB, MAJ, LAN)
        colv = (lax.broadcasted_iota(jnp.int32, (RB, MAJ, LAN), 1) * LAN
                + lax.broadcasted_iota(jnp.int32, (RB, MAJ, LAN), 2)
                + c * CB)
        cand_v0 = jnp.full((RB, K, MAJ), jnp.inf, jnp.float32)
        cand_i0 = jnp.full((RB, K, MAJ), BIG, jnp.int32)

        def cond(st):
            _, _, _, r, done = st
            return jnp.logical_and(r < K, jnp.logical_not(done))

        def body(st):
            dv, cand_v, cand_i, r, _ = st
            m16 = jnp.min(dv, axis=2)
            am16 = jnp.min(jnp.where(dv == m16[:, :, None], colv, BIG), axis=2)
            dv = jnp.where(colv == am16[:, :, None], INF, dv)
            rio = lax.broadcasted_iota(jnp.int32, (RB, K, MAJ), 1)
            cand_v = jnp.where(rio == r, m16[:, None, :], cand_v)
            cand_i = jnp.where(rio == r, am16[:, None, :], cand_i)
            m1 = jnp.min(jnp.min(dv, axis=2), axis=1, keepdims=True)
            cnt = jnp.sum(
                jnp.sum((cand_v < m1[:, :, None]).astype(jnp.int32), axis=2),
                axis=1)
            done = jnp.all(cnt >= K)
            return (dv, cand_v, cand_i, r + 1, done)

        _, cand_v, cand_i, _, _ = lax.while_loop(
            cond, body, (dv, cand_v0, cand_i0, jnp.int32(0), jnp.bool_(False)))

        vals = cand_v.reshape(RB, K * MAJ)
        inds = cand_i.reshape(RB, K * MAJ)
        def step(t, st):
            vals, ov, oi = st
            m = jnp.min(vals, axis=1, keepdims=True)
            am = jnp.min(jnp.where(vals == m, inds, BIG), axis=1, keepdims=True)
            ov = jnp.where(kio == t, m, ov)
            oi = jnp.where(kio == t, am, oi)
            vals = jnp.where(inds == am, INF, vals)
            return (vals, ov, oi)
        _, ov, oi = lax.fori_loop(
            0, K, step,
            (vals, jnp.zeros((RB, K), jnp.float32), jnp.zeros((RB, K), jnp.int32)))
        return ov, oi

    pos32 = lax.broadcasted_iota(jnp.int32, (RB, 2 * K), 1)

    def merge(rv, ri, cv, ci):
        vals = jnp.concatenate([rv, cv], axis=1)
        inds = jnp.concatenate([ri, ci], axis=1)
        def step(t, st):
            vals, ov, oi = st
            m = jnp.min(vals, axis=1, keepdims=True)
            ap = jnp.min(jnp.where(vals == m, pos32, BIG), axis=1, keepdims=True)
            hit = pos32 == ap
            sel = jnp.sum(jnp.where(hit, inds, 0), axis=1, keepdims=True)
            ov = jnp.where(kio == t, m, ov)
            oi = jnp.where(kio == t, sel, oi)
            vals = jnp.where(hit, INF, vals)
            return (vals, ov, oi)
        _, ov, oi = lax.fori_loop(
            0, K, step,
            (vals, jnp.zeros((RB, K), jnp.float32), jnp.zeros((RB, K), jnp.int32)))
        return ov, oi

    def chunk_step(c, st):
        rv, ri = st
        yb = ye_ref[pl.ds(c * CB, CB), :]
        yq = jnp.sum(yb * yb, axis=1)[None, :]
        inner = lax.dot_general(
            xb, yb, (((1,), (1,)), ((), ())),
            preferred_element_type=jnp.float32)
        d2 = jnp.maximum(xq + yq - 2.0 * inner, 0.0)
        d = jnp.sqrt(d2)
        cv, ci = extract_chunk(d, c)
        return merge(rv, ri, cv, ci)

    out_v, out_i = lax.fori_loop(
        0, NCH, chunk_step,
        (jnp.full((RB, K), jnp.inf, jnp.float32),
         jnp.zeros((RB, K), jnp.int32)))

    score_ref[...] = jnp.exp(-out_v / 10.0)
    idx_ref[...] = out_i


def _topk_call(xe2, ye2):
    return pl.pallas_call(
        _topk_body,
        grid=(N // RB,),
        in_specs=[
            pl.BlockSpec((RB, E), lambda i: (i, 0)),
            pl.BlockSpec((N, E), lambda i: (0, 0)),
        ],
        out_specs=[
            pl.BlockSpec((RB, K), lambda i: (i, 0)),
            pl.BlockSpec((RB, K), lambda i: (i, 0)),
            pl.BlockSpec((RB, 128), lambda i: (i, 0)),
        ],
        out_shape=[
            jax.ShapeDtypeStruct((N, K), jnp.float32),
            jax.ShapeDtypeStruct((N, K), jnp.int32),
            jax.ShapeDtypeStruct((N, 128), jnp.float32),
        ],
    )(xe2, ye2)


_B = N * K
_NW = 32
_BPW = _B // _NW
_RC = 512
_GC = 128


def _sc_gather_body(idx_hbm, xe_hbm, out_hbm, idx_v, rows_v, sem):
    wid = lax.axis_index("s") * 2 + lax.axis_index("c")
    base = wid * _BPW

    def outer(t, carry):
        off = base + t * _RC
        pltpu.sync_copy(idx_hbm.at[pl.ds(off, _RC)], idx_v)
        copies = []
        for s in range(_RC // _GC):
            copies.append(pltpu.async_copy(
                xe_hbm.at[idx_v.at[pl.ds(s * _GC, _GC)]],
                rows_v.at[pl.ds(s * _GC, _GC)], sem))
        for c in copies:
            c.wait()
        pltpu.sync_copy(rows_v, out_hbm.at[pl.ds(off, _RC)])
        return carry

    lax.fori_loop(0, _BPW // _RC, outer, 0)


def _sc_gather(idx_flat, xe_pad):
    mesh = plsc.VectorSubcoreMesh(core_axis_name="c", subcore_axis_name="s")
    kfn = functools.partial(
        pl.kernel,
        mesh=mesh,
        out_type=jax.ShapeDtypeStruct((_B, 128), jnp.float32),
        scratch_types=[
            pltpu.VMEM((_RC,), jnp.int32),
            pltpu.VMEM((_RC, 128), jnp.float32),
            pltpu.SemaphoreType.DMA,
        ],
    )(_sc_gather_body)
    return kfn(idx_flat, xe_pad)


def _diff_body(g_ref, ye_ref, out_ref):
    g = g_ref[...][:, :, :E]
    y = ye_ref[...]
    out_ref[...] = y[:, :, None] - jnp.swapaxes(g, 1, 2)


def _diff_call(gath, ye2):
    return pl.pallas_call(
        _diff_body,
        grid=(N // RB,),
        in_specs=[
            pl.BlockSpec((RB, K, 128), lambda i: (i, 0, 0)),
            pl.BlockSpec((RB, E), lambda i: (i, 0)),
        ],
        out_specs=pl.BlockSpec((RB, E, K), lambda i: (i, 0, 0)),
        out_shape=jax.ShapeDtypeStruct((N, E, K), jnp.float32),
    )(gath, ye2)


def kernel(xe, ye, adj_coo):
    del adj_coo
    xe2 = xe[0]
    ye2 = ye[0]
    score, idx, xe_pad = _topk_call(xe2, ye2)
    gath = _sc_gather(idx.reshape(_B), xe_pad)
    diff = _diff_call(gath.reshape(N, K, 128), ye2)
    return score[None], idx[None], diff[None]


# speedup vs baseline: 1.5944x; 1.5944x over previous
import functools

import jax
import jax.numpy as jnp
from jax import lax
from jax.experimental import pallas as pl
from jax.experimental.pallas import tpu as pltpu
from jax.experimental.pallas import tpu_sc as plsc

K = 16
N = 8192
E = 64
RB = 256
CB = 2048
NCH = N // CB


def _topk_body(xe_ref, ye_ref, score_ref, idx_ref, xp_ref):
    xb = xe_ref[...]
    xp_ref[...] = jnp.concatenate(
        [xb, jnp.zeros((RB, 128 - E), jnp.float32)], axis=1)
    xq = jnp.sum(xb * xb, axis=1, keepdims=True)

    BIG = jnp.int32(2 ** 30)
    INF = jnp.float32(jnp.inf)
    kio = lax.broadcasted_iota(jnp.int32, (RB, K), 1)

    H = CB // 2

    def extract_chunk(d, col):
        lo = d[:, :H]
        hi = d[:, H:]
        cl = col[:, :H]
        ch = col[:, H:]
        wl = lo <= hi
        wv = jnp.where(wl, lo, hi)
        wi = jnp.where(wl, cl, ch)
        lv = jnp.where(wl, hi, lo)
        li = jnp.where(wl, ch, cl)

        def step(t, st):
            wv, wi, lv, ov, oi = st
            m = jnp.min(wv, axis=1, keepdims=True)
            am = jnp.min(jnp.where(wv == m, wi, BIG), axis=1, keepdims=True)
            ov = jnp.where(kio == t, m, ov)
            oi = jnp.where(kio == t, am, oi)
            hit = wi == am
            wv = jnp.where(hit, lv, wv)
            wi = jnp.where(hit, li, wi)
            lv = jnp.where(hit, INF, lv)
            return (wv, wi, lv, ov, oi)

        _, _, _, ov, oi = lax.fori_loop(
            0, K, step,
            (wv, wi, lv,
             jnp.zeros((RB, K), jnp.float32), jnp.zeros((RB, K), jnp.int32)))
        return ov, oi

    pos32 = lax.broadcasted_iota(jnp.int32, (RB, 2 * K), 1)

    def merge(rv, ri, cv, ci):
        vals = jnp.concatenate([rv, cv], axis=1)
        inds = jnp.concatenate([ri, ci], axis=1)
        def step(t, st):
            vals, ov, oi = st
            m = jnp.min(vals, axis=1, keepdims=True)
            ap = jnp.min(jnp.where(vals == m, pos32, BIG), axis=1, keepdims=True)
            hit = pos32 == ap
            sel = jnp.sum(jnp.where(hit, inds, 0), axis=1, keepdims=True)
            ov = jnp.where(kio == t, m, ov)
            oi = jnp.where(kio == t, sel, oi)
            vals = jnp.where(hit, INF, vals)
            return (vals, ov, oi)
        _, ov, oi = lax.fori_loop(
            0, K, step,
            (vals, jnp.zeros((RB, K), jnp.float32), jnp.zeros((RB, K), jnp.int32)))
        return ov, oi

    def chunk_step(c, st):
        rv, ri = st
        yb = ye_ref[pl.ds(c * CB, CB), :]
        yq = jnp.sum(yb * yb, axis=1)[None, :]
        inner = lax.dot_general(
            xb, yb, (((1,), (1,)), ((), ())),
            preferred_element_type=jnp.float32)
        d2 = jnp.maximum(xq + yq - 2.0 * inner, 0.0)
        d = jnp.sqrt(d2)
        col = lax.broadcasted_iota(jnp.int32, (RB, CB), 1) + c * CB
        cv, ci = extract_chunk(d, col)
        return merge(rv, ri, cv, ci)

    out_v, out_i = lax.fori_loop(
        0, NCH, chunk_step,
        (jnp.full((RB, K), jnp.inf, jnp.float32),
         jnp.zeros((RB, K), jnp.int32)))

    score_ref[...] = jnp.exp(-out_v / 10.0)
    idx_ref[...] = out_i


def _topk_call(xe2, ye2):
    return pl.pallas_call(
        _topk_body,
        grid=(N // RB,),
        in_specs=[
            pl.BlockSpec((RB, E), lambda i: (i, 0)),
            pl.BlockSpec((N, E), lambda i: (0, 0)),
        ],
        out_specs=[
            pl.BlockSpec((RB, K), lambda i: (i, 0)),
            pl.BlockSpec((RB, K), lambda i: (i, 0)),
            pl.BlockSpec((RB, 128), lambda i: (i, 0)),
        ],
        out_shape=[
            jax.ShapeDtypeStruct((N, K), jnp.float32),
            jax.ShapeDtypeStruct((N, K), jnp.int32),
            jax.ShapeDtypeStruct((N, 128), jnp.float32),
        ],
    )(xe2, ye2)


_B = N * K
_NW = 32
_BPW = _B // _NW
_RC = 512
_GC = 128


def _sc_gather_body(idx_hbm, xe_hbm, out_hbm, idx_v, rows_v, sem):
    wid = lax.axis_index("s") * 2 + lax.axis_index("c")
    base = wid * _BPW

    def outer(t, carry):
        off = base + t * _RC
        pltpu.sync_copy(idx_hbm.at[pl.ds(off, _RC)], idx_v)
        copies = []
        for s in range(_RC // _GC):
            copies.append(pltpu.async_copy(
                xe_hbm.at[idx_v.at[pl.ds(s * _GC, _GC)]],
                rows_v.at[pl.ds(s * _GC, _GC)], sem))
        for c in copies:
            c.wait()
        pltpu.sync_copy(rows_v, out_hbm.at[pl.ds(off, _RC)])
        return carry

    lax.fori_loop(0, _BPW // _RC, outer, 0)


def _sc_gather(idx_flat, xe_pad):
    mesh = plsc.VectorSubcoreMesh(core_axis_name="c", subcore_axis_name="s")
    kfn = functools.partial(
        pl.kernel,
        mesh=mesh,
        out_type=jax.ShapeDtypeStruct((_B, 128), jnp.float32),
        scratch_types=[
            pltpu.VMEM((_RC,), jnp.int32),
            pltpu.VMEM((_RC, 128), jnp.float32),
            pltpu.SemaphoreType.DMA,
        ],
    )(_sc_gather_body)
    return kfn(idx_flat, xe_pad)


def _diff_body(g_ref, ye_ref, out_ref):
    g = g_ref[...][:, :, :E]
    y = ye_ref[...]
    out_ref[...] = y[:, :, None] - jnp.swapaxes(g, 1, 2)


def _diff_call(gath, ye2):
    return pl.pallas_call(
        _diff_body,
        grid=(N // RB,),
        in_specs=[
            pl.BlockSpec((RB, K, 128), lambda i: (i, 0, 0)),
            pl.BlockSpec((RB, E), lambda i: (i, 0)),
        ],
        out_specs=pl.BlockSpec((RB, E, K), lambda i: (i, 0, 0)),
        out_shape=jax.ShapeDtypeStruct((N, E, K), jnp.float32),
    )(gath, ye2)


def kernel(xe, ye, adj_coo):
    del adj_coo
    xe2 = xe[0]
    ye2 = ye[0]
    score, idx, xe_pad = _topk_call(xe2, ye2)
    gath = _sc_gather(idx.reshape(_B), xe_pad)
    diff = _diff_call(gath.reshape(N, K, 128), ye2)
    return score[None], idx[None], diff[None]


# speedup vs baseline: 2.0192x; 1.2664x over previous
import functools

import jax
import jax.numpy as jnp
from jax import lax
from jax.experimental import pallas as pl
from jax.experimental.pallas import tpu as pltpu
from jax.experimental.pallas import tpu_sc as plsc

K = 16
N = 8192
E = 64
RB = 256
CB = 2048
NCH = N // CB


def _topk_body(xe_ref, ye_ref, score_ref, idx_ref, xp_ref):
    xb = xe_ref[...]
    xp_ref[...] = jnp.concatenate(
        [xb, jnp.zeros((RB, 128 - E), jnp.float32)], axis=1)
    xq = jnp.sum(xb * xb, axis=1, keepdims=True)

    BIG = jnp.int32(2 ** 30)
    INF = jnp.float32(jnp.inf)
    kio = lax.broadcasted_iota(jnp.int32, (RB, K), 1)

    SS = 4

    def extract_chunk(d, col):
        ov = jnp.zeros((RB, K), jnp.float32)
        oi = jnp.zeros((RB, K), jnp.int32)
        for s in range(K // SS):
            ams = []
            for j in range(SS):
                dm = d
                for am_prev in ams:
                    dm = jnp.where(col == am_prev, INF, dm)
                m = jnp.min(dm, axis=1, keepdims=True)
                am = jnp.min(jnp.where(dm == m, col, BIG), axis=1, keepdims=True)
                t = s * SS + j
                ov = jnp.where(kio == t, m, ov)
                oi = jnp.where(kio == t, am, oi)
                ams.append(am)
            hit = col == ams[0]
            for am_prev in ams[1:]:
                hit = jnp.logical_or(hit, col == am_prev)
            d = jnp.where(hit, INF, d)
        return ov, oi

    pos32 = lax.broadcasted_iota(jnp.int32, (RB, 2 * K), 1)

    def merge(rv, ri, cv, ci):
        vals = jnp.concatenate([rv, cv], axis=1)
        inds = jnp.concatenate([ri, ci], axis=1)
        def step(t, st):
            vals, ov, oi = st
            m = jnp.min(vals, axis=1, keepdims=True)
            ap = jnp.min(jnp.where(vals == m, pos32, BIG), axis=1, keepdims=True)
            hit = pos32 == ap
            sel = jnp.sum(jnp.where(hit, inds, 0), axis=1, keepdims=True)
            ov = jnp.where(kio == t, m, ov)
            oi = jnp.where(kio == t, sel, oi)
            vals = jnp.where(hit, INF, vals)
            return (vals, ov, oi)
        _, ov, oi = lax.fori_loop(
            0, K, step,
            (vals, jnp.zeros((RB, K), jnp.float32), jnp.zeros((RB, K), jnp.int32)))
        return ov, oi

    def chunk_step(c, st):
        rv, ri = st
        yb = ye_ref[pl.ds(c * CB, CB), :]
        yq = jnp.sum(yb * yb, axis=1)[None, :]
        inner = lax.dot_general(
            xb, yb, (((1,), (1,)), ((), ())),
            preferred_element_type=jnp.float32)
        d2 = jnp.maximum(xq + yq - 2.0 * inner, 0.0)
        d = jnp.sqrt(d2)
        col = lax.broadcasted_iota(jnp.int32, (RB, CB), 1) + c * CB
        cv, ci = extract_chunk(d, col)
        return merge(rv, ri, cv, ci)

    out_v, out_i = lax.fori_loop(
        0, NCH, chunk_step,
        (jnp.full((RB, K), jnp.inf, jnp.float32),
         jnp.zeros((RB, K), jnp.int32)))

    score_ref[...] = jnp.exp(-out_v / 10.0)
    idx_ref[...] = out_i


def _topk_call(xe2, ye2):
    return pl.pallas_call(
        _topk_body,
        grid=(N // RB,),
        in_specs=[
            pl.BlockSpec((RB, E), lambda i: (i, 0)),
            pl.BlockSpec((N, E), lambda i: (0, 0)),
        ],
        out_specs=[
            pl.BlockSpec((RB, K), lambda i: (i, 0)),
            pl.BlockSpec((RB, K), lambda i: (i, 0)),
            pl.BlockSpec((RB, 128), lambda i: (i, 0)),
        ],
        out_shape=[
            jax.ShapeDtypeStruct((N, K), jnp.float32),
            jax.ShapeDtypeStruct((N, K), jnp.int32),
            jax.ShapeDtypeStruct((N, 128), jnp.float32),
        ],
    )(xe2, ye2)


_B = N * K
_NW = 32
_BPW = _B // _NW
_RC = 512
_GC = 128


def _sc_gather_body(idx_hbm, xe_hbm, out_hbm, idx_v, rows_v, sem):
    wid = lax.axis_index("s") * 2 + lax.axis_index("c")
    base = wid * _BPW

    def outer(t, carry):
        off = base + t * _RC
        pltpu.sync_copy(idx_hbm.at[pl.ds(off, _RC)], idx_v)
        copies = []
        for s in range(_RC // _GC):
            copies.append(pltpu.async_copy(
                xe_hbm.at[idx_v.at[pl.ds(s * _GC, _GC)]],
                rows_v.at[pl.ds(s * _GC, _GC)], sem))
        for c in copies:
            c.wait()
        pltpu.sync_copy(rows_v, out_hbm.at[pl.ds(off, _RC)])
        return carry

    lax.fori_loop(0, _BPW // _RC, outer, 0)


def _sc_gather(idx_flat, xe_pad):
    mesh = plsc.VectorSubcoreMesh(core_axis_name="c", subcore_axis_name="s")
    kfn = functools.partial(
        pl.kernel,
        mesh=mesh,
        out_type=jax.ShapeDtypeStruct((_B, 128), jnp.float32),
        scratch_types=[
            pltpu.VMEM((_RC,), jnp.int32),
            pltpu.VMEM((_RC, 128), jnp.float32),
            pltpu.SemaphoreType.DMA,
        ],
    )(_sc_gather_body)
    return kfn(idx_flat, xe_pad)


def _diff_body(g_ref, ye_ref, out_ref):
    g = g_ref[...][:, :, :E]
    y = ye_ref[...]
    out_ref[...] = y[:, :, None] - jnp.swapaxes(g, 1, 2)


def _diff_call(gath, ye2):
    return pl.pallas_call(
        _diff_body,
        grid=(N // RB,),
        in_specs=[
            pl.BlockSpec((RB, K, 128), lambda i: (i, 0, 0)),
            pl.BlockSpec((RB, E), lambda i: (i, 0)),
        ],
        out_specs=pl.BlockSpec((RB, E, K), lambda i: (i, 0, 0)),
        out_shape=jax.ShapeDtypeStruct((N, E, K), jnp.float32),
    )(gath, ye2)


def kernel(xe, ye, adj_coo):
    del adj_coo
    xe2 = xe[0]
    ye2 = ye[0]
    score, idx, xe_pad = _topk_call(xe2, ye2)
    gath = _sc_gather(idx.reshape(_B), xe_pad)
    diff = _diff_call(gath.reshape(N, K, 128), ye2)
    return score[None], idx[None], diff[None]


# speedup vs baseline: 2.0243x; 1.0025x over previous
import functools

import jax
import jax.numpy as jnp
from jax import lax
from jax.experimental import pallas as pl
from jax.experimental.pallas import tpu as pltpu
from jax.experimental.pallas import tpu_sc as plsc

K = 16
N = 8192
E = 64
RB = 256
CB = 2048
NCH = N // CB


def _topk_body(xe_ref, ye_ref, score_ref, idx_ref, xp_ref):
    xb = xe_ref[...]
    xp_ref[...] = jnp.concatenate(
        [xb, jnp.zeros((RB, 128 - E), jnp.float32)], axis=1)
    xq = jnp.sum(xb * xb, axis=1, keepdims=True)

    BIG = jnp.int32(2 ** 30)
    INF = jnp.float32(jnp.inf)
    kio = lax.broadcasted_iota(jnp.int32, (RB, K), 1)

    SS = 8

    def extract_chunk(d, col):
        ov = jnp.zeros((RB, K), jnp.float32)
        oi = jnp.zeros((RB, K), jnp.int32)
        for s in range(K // SS):
            ams = []
            for j in range(SS):
                dm = d
                for am_prev in ams:
                    dm = jnp.where(col == am_prev, INF, dm)
                m = jnp.min(dm, axis=1, keepdims=True)
                am = jnp.min(jnp.where(dm == m, col, BIG), axis=1, keepdims=True)
                t = s * SS + j
                ov = jnp.where(kio == t, m, ov)
                oi = jnp.where(kio == t, am, oi)
                ams.append(am)
            hit = col == ams[0]
            for am_prev in ams[1:]:
                hit = jnp.logical_or(hit, col == am_prev)
            d = jnp.where(hit, INF, d)
        return ov, oi

    pos32 = lax.broadcasted_iota(jnp.int32, (RB, 2 * K), 1)

    def merge(rv, ri, cv, ci):
        vals = jnp.concatenate([rv, cv], axis=1)
        inds = jnp.concatenate([ri, ci], axis=1)
        def step(t, st):
            vals, ov, oi = st
            m = jnp.min(vals, axis=1, keepdims=True)
            ap = jnp.min(jnp.where(vals == m, pos32, BIG), axis=1, keepdims=True)
            hit = pos32 == ap
            sel = jnp.sum(jnp.where(hit, inds, 0), axis=1, keepdims=True)
            ov = jnp.where(kio == t, m, ov)
            oi = jnp.where(kio == t, sel, oi)
            vals = jnp.where(hit, INF, vals)
            return (vals, ov, oi)
        _, ov, oi = lax.fori_loop(
            0, K, step,
            (vals, jnp.zeros((RB, K), jnp.float32), jnp.zeros((RB, K), jnp.int32)))
        return ov, oi

    def chunk_step(c, st):
        rv, ri = st
        yb = ye_ref[pl.ds(c * CB, CB), :]
        yq = jnp.sum(yb * yb, axis=1)[None, :]
        inner = lax.dot_general(
            xb, yb, (((1,), (1,)), ((), ())),
            preferred_element_type=jnp.float32)
        d2 = jnp.maximum(xq + yq - 2.0 * inner, 0.0)
        d = jnp.sqrt(d2)
        col = lax.broadcasted_iota(jnp.int32, (RB, CB), 1) + c * CB
        cv, ci = extract_chunk(d, col)
        return merge(rv, ri, cv, ci)

    out_v, out_i = lax.fori_loop(
        0, NCH, chunk_step,
        (jnp.full((RB, K), jnp.inf, jnp.float32),
         jnp.zeros((RB, K), jnp.int32)))

    score_ref[...] = jnp.exp(-out_v / 10.0)
    idx_ref[...] = out_i


def _topk_call(xe2, ye2):
    return pl.pallas_call(
        _topk_body,
        grid=(N // RB,),
        in_specs=[
            pl.BlockSpec((RB, E), lambda i: (i, 0)),
            pl.BlockSpec((N, E), lambda i: (0, 0)),
        ],
        out_specs=[
            pl.BlockSpec((RB, K), lambda i: (i, 0)),
            pl.BlockSpec((RB, K), lambda i: (i, 0)),
            pl.BlockSpec((RB, 128), lambda i: (i, 0)),
        ],
        out_shape=[
            jax.ShapeDtypeStruct((N, K), jnp.float32),
            jax.ShapeDtypeStruct((N, K), jnp.int32),
            jax.ShapeDtypeStruct((N, 128), jnp.float32),
        ],
    )(xe2, ye2)


_B = N * K
_NW = 32
_BPW = _B // _NW
_RC = 512
_GC = 128


def _sc_gather_body(idx_hbm, xe_hbm, out_hbm, idx_v, rows_v, sem):
    wid = lax.axis_index("s") * 2 + lax.axis_index("c")
    base = wid * _BPW

    def outer(t, carry):
        off = base + t * _RC
        pltpu.sync_copy(idx_hbm.at[pl.ds(off, _RC)], idx_v)
        copies = []
        for s in range(_RC // _GC):
            copies.append(pltpu.async_copy(
                xe_hbm.at[idx_v.at[pl.ds(s * _GC, _GC)]],
                rows_v.at[pl.ds(s * _GC, _GC)], sem))
        for c in copies:
            c.wait()
        pltpu.sync_copy(rows_v, out_hbm.at[pl.ds(off, _RC)])
        return carry

    lax.fori_loop(0, _BPW // _RC, outer, 0)


def _sc_gather(idx_flat, xe_pad):
    mesh = plsc.VectorSubcoreMesh(core_axis_name="c", subcore_axis_name="s")
    kfn = functools.partial(
        pl.kernel,
        mesh=mesh,
        out_type=jax.ShapeDtypeStruct((_B, 128), jnp.float32),
        scratch_types=[
            pltpu.VMEM((_RC,), jnp.int32),
            pltpu.VMEM((_RC, 128), jnp.float32),
            pltpu.SemaphoreType.DMA,
        ],
    )(_sc_gather_body)
    return kfn(idx_flat, xe_pad)


def _diff_body(g_ref, ye_ref, out_ref):
    g = g_ref[...][:, :, :E]
    y = ye_ref[...]
    out_ref[...] = y[:, :, None] - jnp.swapaxes(g, 1, 2)


def _diff_call(gath, ye2):
    return pl.pallas_call(
        _diff_body,
        grid=(N // RB,),
        in_specs=[
            pl.BlockSpec((RB, K, 128), lambda i: (i, 0, 0)),
            pl.BlockSpec((RB, E), lambda i: (i, 0)),
        ],
        out_specs=pl.BlockSpec((RB, E, K), lambda i: (i, 0, 0)),
        out_shape=jax.ShapeDtypeStruct((N, E, K), jnp.float32),
    )(gath, ye2)


def kernel(xe, ye, adj_coo):
    del adj_coo
    xe2 = xe[0]
    ye2 = ye[0]
    score, idx, xe_pad = _topk_call(xe2, ye2)
    gath = _sc_gather(idx.reshape(_B), xe_pad)
    diff = _diff_call(gath.reshape(N, K, 128), ye2)
    return score[None], idx[None], diff[None]
